# baseline (speedup 1.0000x reference)
import functools

import jax
import jax.numpy as jnp
from jax import lax
from jax.experimental import pallas as pl
from jax.experimental.pallas import tpu as pltpu
from jax.experimental.pallas import tpu_sc as plsc

NC = 2
NS = 16
NW = NC * NS
L = 16

CH = 128


def _rsqrt_nr(x):
    xi = plsc.bitcast(x, jnp.int32)
    yi = jnp.int32(0x5F3759DF) - (xi >> 1)
    y = plsc.bitcast(yi, jnp.float32)
    for _ in range(3):
        y = y * (jnp.float32(1.5) - jnp.float32(0.5) * x * y * y)
    return y


def _make_sc_call(B, D):
    b_per_w = B // NW
    nch = b_per_w // CH
    groups = b_per_w // L
    mesh = plsc.VectorSubcoreMesh(
        core_axis_name="c", subcore_axis_name="s", num_cores=NC, num_subcores=NS
    )

    @functools.partial(
        pl.kernel,
        out_type=jax.ShapeDtypeStruct((B,), jnp.float32),
        mesh=mesh,
        compiler_params=pltpu.CompilerParams(
            needs_layout_passes=False, use_tc_tiling_on_sc=False),
        scratch_types=[
            pltpu.VMEM((nch, CH), jnp.int32),
            pltpu.VMEM((nch, CH), jnp.int32),
            pltpu.VMEM((b_per_w, D), jnp.float32),
            pltpu.VMEM((b_per_w, D), jnp.float32),
            pltpu.VMEM((b_per_w,), jnp.float32),
            pltpu.SemaphoreType.DMA,
            pltpu.SemaphoreType.DMA,
        ],
    )
    def sc_call(uid_hbm, iid_hbm, ut_hbm, it_hbm, out_hbm,
                uidx_v, iidx_v, urows_v, irows_v, res_v, usem, isem):
        wid = lax.axis_index("s") * NC + lax.axis_index("c")
        base = wid * b_per_w

        pltpu.sync_copy(uid_hbm.at[wid], uidx_v)
        pltpu.sync_copy(iid_hbm.at[wid], iidx_v)

        ucopies = []
        icopies = []
        for j in range(nch):
            ucopies.append(pltpu.async_copy(
                ut_hbm.at[uidx_v.at[j]],
                urows_v.at[pl.ds(j * CH, CH), :], usem))
            icopies.append(pltpu.async_copy(
                it_hbm.at[iidx_v.at[j]],
                irows_v.at[pl.ds(j * CH, CH), :], isem))
        for c in ucopies + icopies:
            c.wait()

        def group_body(g, _):
            rows = g * L + lax.iota(jnp.int32, L)
            dot = jnp.zeros((L,), jnp.float32)
            nu2 = jnp.zeros((L,), jnp.float32)
            ni2 = jnp.zeros((L,), jnp.float32)
            for d in range(D):
                col = jnp.full((L,), d, jnp.int32)
                u = plsc.load_gather(urows_v, [rows, col])
                v = plsc.load_gather(irows_v, [rows, col])
                dot = dot + u * v
                nu2 = nu2 + u * u
                ni2 = ni2 + v * v
            rnu = _rsqrt_nr(jnp.maximum(nu2, jnp.float32(1e-16)))
            rni = _rsqrt_nr(jnp.maximum(ni2, jnp.float32(1e-16)))
            res_v[pl.ds(g * L, L)] = dot * rnu * rni
            return 0

        lax.fori_loop(0, groups, group_body, 0)
        pltpu.sync_copy(res_v, out_hbm.at[pl.ds(base, b_per_w)])

    return sc_call


def kernel(user_id, item_id, user_table, item_table):
    B = user_id.shape[0]
    D = user_table.shape[1]
    uid = user_id.astype(jnp.int32).reshape(NW, B // NW // CH, CH)
    iid = item_id.astype(jnp.int32).reshape(NW, B // NW // CH, CH)
    return _make_sc_call(B, D)(uid, iid, user_table, item_table)
